# baseline (speedup 1.0000x reference)
import functools

import jax
import jax.numpy as jnp
from jax import lax
from jax.experimental import pallas as pl
from jax.experimental.pallas import tpu as pltpu
from jax.experimental.pallas import tpu_sc as plsc

N_NODES = 10000
N_EDGES = 160000
IN_DIM = 1152
HID_DIM = 768
EDGE_DIM = 16
EDGE_HID = 64

_M_BLK = 1000


def _node_body(nf, fii, wa, wb, w1, w2, fii_out, h1_out, h2_out):
    x = nf[...]
    a = jnp.dot(x, wa[...], preferred_element_type=jnp.float32)
    b = jnp.dot(x, wb[...], preferred_element_type=jnp.float32)
    fii_out[...] = fii[...] + a * jax.nn.sigmoid(a) * b
    h1_out[...] = jnp.dot(x, w1[...], preferred_element_type=jnp.float32)
    h2_out[...] = jnp.dot(x, w2[...], preferred_element_type=jnp.float32)


def _node_kernel(node_feat, fii, wa, wb, w1, w2):
    grid = (N_NODES // _M_BLK,)
    out_shape = [jax.ShapeDtypeStruct((N_NODES, HID_DIM), jnp.float32)] * 3
    return pl.pallas_call(
        _node_body,
        grid=grid,
        in_specs=[
            pl.BlockSpec((_M_BLK, IN_DIM), lambda i: (i, 0)),
            pl.BlockSpec((_M_BLK, HID_DIM), lambda i: (i, 0)),
            pl.BlockSpec((IN_DIM, HID_DIM), lambda i: (0, 0)),
            pl.BlockSpec((IN_DIM, HID_DIM), lambda i: (0, 0)),
            pl.BlockSpec((IN_DIM, HID_DIM), lambda i: (0, 0)),
            pl.BlockSpec((IN_DIM, HID_DIM), lambda i: (0, 0)),
        ],
        out_specs=[pl.BlockSpec((_M_BLK, HID_DIM), lambda i: (i, 0))] * 3,
        out_shape=out_shape,
    )(node_feat, fii, wa, wb, w1, w2)


_E_BLK = 2000


def _gate_body(ea, we1, be1, we2, be2, gate_out):
    t = jnp.dot(ea[...], we1[...], preferred_element_type=jnp.float32) + be1[...]
    t = t * jax.nn.sigmoid(t)
    gate_out[...] = (
        jnp.dot(t, we2[...], preferred_element_type=jnp.float32) + be2[...]
    )


def _gate_kernel(edge_attr, we1, be1, we2, be2):
    grid = (N_EDGES // _E_BLK,)
    return pl.pallas_call(
        _gate_body,
        grid=grid,
        in_specs=[
            pl.BlockSpec((_E_BLK, EDGE_DIM), lambda i: (i, 0)),
            pl.BlockSpec((EDGE_DIM, EDGE_HID), lambda i: (0, 0)),
            pl.BlockSpec((1, EDGE_HID), lambda i: (0, 0)),
            pl.BlockSpec((EDGE_HID, HID_DIM), lambda i: (0, 0)),
            pl.BlockSpec((1, HID_DIM), lambda i: (0, 0)),
        ],
        out_specs=pl.BlockSpec((_E_BLK, HID_DIM), lambda i: (i, 0)),
        out_shape=jax.ShapeDtypeStruct((N_EDGES, HID_DIM), jnp.float32),
    )(edge_attr, we1, be1, we2, be2)


_NC = 2
_NS = 16
_NW = _NC * _NS
_E_PER_W = N_EDGES // _NW
_B = 40
_N_CHUNKS = _E_PER_W // _B
_LANES = 16
_COLS = HID_DIM // _LANES


def _edge_body(h1_hbm, h2_hbm, gate_hbm, fij_hbm, src_hbm, dst_hbm, out_hbm,
               src_v, dst_v, r1, r2, gv, fv, sem1, sem2, sem3, sem4):
    wid = lax.axis_index("s") * _NC + lax.axis_index("c")
    base0 = wid * _E_PER_W

    def chunk_body(c, carry):
        base = base0 + c * _B
        pltpu.sync_copy(src_hbm.at[pl.ds(base, _B)], src_v)
        pltpu.sync_copy(dst_hbm.at[pl.ds(base, _B)], dst_v)
        cp1 = pltpu.make_async_copy(h1_hbm.at[src_v], r1, sem1)
        cp2 = pltpu.make_async_copy(h2_hbm.at[dst_v], r2, sem2)
        cp3 = pltpu.make_async_copy(gate_hbm.at[pl.ds(base, _B), :], gv, sem3)
        cp4 = pltpu.make_async_copy(fij_hbm.at[pl.ds(base, _B), :], fv, sem4)
        cp1.start()
        cp2.start()
        cp3.start()
        cp4.start()
        cp1.wait()
        cp2.wait()
        cp3.wait()
        cp4.wait()

        def row_body(i, carry2):
            def col_body(j, carry3):
                sl = pl.ds(j * _LANES, _LANES)
                fv[i, sl] = fv[i, sl] + r1[i, sl] * r2[i, sl] * gv[i, sl]
                return carry3

            return lax.fori_loop(0, _COLS, col_body, carry2)

        lax.fori_loop(0, _B, row_body, 0)
        pltpu.sync_copy(fv, out_hbm.at[pl.ds(base, _B), :])
        return carry

    lax.fori_loop(0, _N_CHUNKS, chunk_body, 0)


def _edge_kernel(h1, h2, gate, fij, src, dst):
    mesh = plsc.VectorSubcoreMesh(core_axis_name="c", subcore_axis_name="s")
    f = functools.partial(
        pl.kernel,
        out_type=jax.ShapeDtypeStruct((N_EDGES, HID_DIM), jnp.float32),
        mesh=mesh,
        scratch_types=[
            pltpu.VMEM((_B,), jnp.int32),
            pltpu.VMEM((_B,), jnp.int32),
            pltpu.VMEM((_B, HID_DIM), jnp.float32),
            pltpu.VMEM((_B, HID_DIM), jnp.float32),
            pltpu.VMEM((_B, HID_DIM), jnp.float32),
            pltpu.VMEM((_B, HID_DIM), jnp.float32),
            pltpu.SemaphoreType.DMA,
            pltpu.SemaphoreType.DMA,
            pltpu.SemaphoreType.DMA,
            pltpu.SemaphoreType.DMA,
        ],
    )(_edge_body)
    return f(h1, h2, gate, fij, src, dst)


def kernel(node_feat, edge_attr, edge_index, fii, fij,
           W_self_a, W_self_b, W1, W2, We1, be1, We2, be2):
    src = edge_index[0].astype(jnp.int32)
    dst = edge_index[1].astype(jnp.int32)
    fii_out, h1, h2 = _node_kernel(node_feat, fii, W_self_a, W_self_b, W1, W2)
    gate = _gate_kernel(
        edge_attr, We1, be1.reshape(1, EDGE_HID), We2, be2.reshape(1, HID_DIM)
    )
    fij_out = _edge_kernel(h1, h2, gate, fij, src, dst)
    return (fii_out, fij_out)


# speedup vs baseline: 2.0783x; 2.0783x over previous
import functools

import jax
import jax.numpy as jnp
from jax import lax
from jax.experimental import pallas as pl
from jax.experimental.pallas import tpu as pltpu
from jax.experimental.pallas import tpu_sc as plsc

N_NODES = 10000
N_EDGES = 160000
IN_DIM = 1152
HID_DIM = 768
EDGE_DIM = 16
EDGE_HID = 64

_M_BLK = 1000


def _node_body(nf, fii, wa, wb, w1, w2, fii_out, h1_out, h2_out):
    x = nf[...]
    a = jnp.dot(x, wa[...], preferred_element_type=jnp.float32)
    b = jnp.dot(x, wb[...], preferred_element_type=jnp.float32)
    fii_out[...] = fii[...] + a * jax.nn.sigmoid(a) * b
    h1_out[...] = jnp.dot(x, w1[...], preferred_element_type=jnp.float32)
    h2_out[...] = jnp.dot(x, w2[...], preferred_element_type=jnp.float32)


def _node_kernel(node_feat, fii, wa, wb, w1, w2):
    grid = (N_NODES // _M_BLK,)
    out_shape = [jax.ShapeDtypeStruct((N_NODES, HID_DIM), jnp.float32)] * 3
    return pl.pallas_call(
        _node_body,
        grid=grid,
        in_specs=[
            pl.BlockSpec((_M_BLK, IN_DIM), lambda i: (i, 0)),
            pl.BlockSpec((_M_BLK, HID_DIM), lambda i: (i, 0)),
            pl.BlockSpec((IN_DIM, HID_DIM), lambda i: (0, 0)),
            pl.BlockSpec((IN_DIM, HID_DIM), lambda i: (0, 0)),
            pl.BlockSpec((IN_DIM, HID_DIM), lambda i: (0, 0)),
            pl.BlockSpec((IN_DIM, HID_DIM), lambda i: (0, 0)),
        ],
        out_specs=[pl.BlockSpec((_M_BLK, HID_DIM), lambda i: (i, 0))] * 3,
        out_shape=out_shape,
    )(node_feat, fii, wa, wb, w1, w2)


_E_BLK = 2000


def _gate_body(ea, we1, be1, we2, be2, gate_out):
    t = jnp.dot(ea[...], we1[...], preferred_element_type=jnp.float32) + be1[...]
    t = t * jax.nn.sigmoid(t)
    gate_out[...] = (
        jnp.dot(t, we2[...], preferred_element_type=jnp.float32) + be2[...]
    )


def _gate_kernel(edge_attr, we1, be1, we2, be2):
    grid = (N_EDGES // _E_BLK,)
    return pl.pallas_call(
        _gate_body,
        grid=grid,
        in_specs=[
            pl.BlockSpec((_E_BLK, EDGE_DIM), lambda i: (i, 0)),
            pl.BlockSpec((EDGE_DIM, EDGE_HID), lambda i: (0, 0)),
            pl.BlockSpec((1, EDGE_HID), lambda i: (0, 0)),
            pl.BlockSpec((EDGE_HID, HID_DIM), lambda i: (0, 0)),
            pl.BlockSpec((1, HID_DIM), lambda i: (0, 0)),
        ],
        out_specs=pl.BlockSpec((_E_BLK, HID_DIM), lambda i: (i, 0)),
        out_shape=jax.ShapeDtypeStruct((N_EDGES, HID_DIM), jnp.float32),
    )(edge_attr, we1, be1, we2, be2)


_NC = 2
_NS = 16
_NW = _NC * _NS
_E_PER_W = N_EDGES // _NW
_B = 8
_NBUF = 4
_N_CHUNKS = _E_PER_W // _B
_N_GROUPS = (_N_CHUNKS + _NBUF - 1) // _NBUF
_LANES = 16
_COLS = HID_DIM // _LANES


def _edge_body(h1_hbm, h2_hbm, gate_hbm, fij_hbm, src_hbm, dst_hbm, out_hbm,
               src_all, dst_all, r1, r2, gv, fv,
               si0, si1, si2, si3, so0, so1, so2, so3):
    sem_in = [si0, si1, si2, si3]
    sem_out = [so0, so1, so2, so3]
    wid = lax.axis_index("s") * _NC + lax.axis_index("c")
    base0 = wid * _E_PER_W
    pltpu.sync_copy(src_hbm.at[pl.ds(base0, _E_PER_W)], src_all)
    pltpu.sync_copy(dst_hbm.at[pl.ds(base0, _E_PER_W)], dst_all)

    def in_copies(c, b):
        base = base0 + c * _B
        return [
            pltpu.make_async_copy(
                h1_hbm.at[src_all.at[pl.ds(c * _B, _B)]], r1.at[b], sem_in[b]),
            pltpu.make_async_copy(
                h2_hbm.at[dst_all.at[pl.ds(c * _B, _B)]], r2.at[b], sem_in[b]),
            pltpu.make_async_copy(
                gate_hbm.at[pl.ds(base, _B), :], gv.at[b], sem_in[b]),
            pltpu.make_async_copy(
                fij_hbm.at[pl.ds(base, _B), :], fv.at[b], sem_in[b]),
        ]

    def out_copy(c, b):
        base = base0 + c * _B
        return pltpu.make_async_copy(
            fv.at[b], out_hbm.at[pl.ds(base, _B), :], sem_out[b])

    for b in range(_NBUF - 1):
        for cp in in_copies(b, b):
            cp.start()

    def group_body(g, carry):
        for b in range(_NBUF):
            c = g * _NBUF + b

            @pl.when(c < _N_CHUNKS)
            def _():
                for cp in in_copies(c, b):
                    cp.wait()

                def row_body(i, carry2):
                    for j in range(_COLS):
                        sl = pl.ds(j * _LANES, _LANES)
                        t = r1[b, i, sl] * r2[b, i, sl] * gv[b, i, sl]
                        plsc.addupdate(fv.at[b, i, sl], t)
                    return carry2

                lax.fori_loop(0, _B, row_body, 0)
                out_copy(c, b).start()

            c2 = c + _NBUF - 1
            b2 = (b + _NBUF - 1) % _NBUF

            @pl.when((c2 >= _NBUF) & (c2 < _N_CHUNKS))
            def _():
                out_copy(c2 - _NBUF, b2).wait()

            @pl.when(c2 < _N_CHUNKS)
            def _():
                for cp in in_copies(c2, b2):
                    cp.start()

        return carry

    lax.fori_loop(0, _N_GROUPS, group_body, 0)
    for b in range(_NBUF):
        last_c = _N_CHUNKS - 1 - ((_N_CHUNKS - 1 - b) % _NBUF)
        out_copy(last_c, b).wait()


def _edge_kernel(h1, h2, gate, fij, src, dst):
    mesh = plsc.VectorSubcoreMesh(core_axis_name="c", subcore_axis_name="s")
    f = functools.partial(
        pl.kernel,
        out_type=jax.ShapeDtypeStruct((N_EDGES, HID_DIM), jnp.float32),
        mesh=mesh,
        scratch_types=[
            pltpu.VMEM((_E_PER_W,), jnp.int32),
            pltpu.VMEM((_E_PER_W,), jnp.int32),
            pltpu.VMEM((_NBUF, _B, HID_DIM), jnp.float32),
            pltpu.VMEM((_NBUF, _B, HID_DIM), jnp.float32),
            pltpu.VMEM((_NBUF, _B, HID_DIM), jnp.float32),
            pltpu.VMEM((_NBUF, _B, HID_DIM), jnp.float32),
        ] + [pltpu.SemaphoreType.DMA] * (2 * _NBUF),
    )(_edge_body)
    return f(h1, h2, gate, fij, src, dst)


def kernel(node_feat, edge_attr, edge_index, fii, fij,
           W_self_a, W_self_b, W1, W2, We1, be1, We2, be2):
    src = edge_index[0].astype(jnp.int32)
    dst = edge_index[1].astype(jnp.int32)
    fii_out, h1, h2 = _node_kernel(node_feat, fii, W_self_a, W_self_b, W1, W2)
    gate = _gate_kernel(
        edge_attr, We1, be1.reshape(1, EDGE_HID), We2, be2.reshape(1, HID_DIM)
    )
    fij_out = _edge_kernel(h1, h2, gate, fij, src, dst)
    return (fii_out, fij_out)
